# Initial kernel scaffold; baseline (speedup 1.0000x reference)
import jax
import jax.numpy as jnp
from jax.experimental import pallas as pl


def kernel(n, relative_attention_bias):
    raise NotImplementedError("write your pallas kernel here")


# speedup vs baseline: 37.2353x; 37.2353x over previous
import functools
import math

import jax
import jax.numpy as jnp
from jax import lax
from jax.experimental import pallas as pl
from jax.experimental.pallas import tpu as pltpu
from jax.experimental.pallas import tpu_sc as plsc

N = 2048
H = 16
NBUCKETS = 32
MAX_DISTANCE = 128
SH = 16
LW = 4352
LEXT = 4480

_ROWS_PER_TILE = N // 32
_FLIGHT = 8


def _prep_body(table_t_ref, out_ref):
    u = lax.broadcasted_iota(jnp.int32, (1, LEXT), 1)
    nv = (N - 1) - u
    neg = jnp.where(nv < 0, NBUCKETS // 2, 0)
    a = jnp.abs(nv)
    small = a < (NBUCKETS // 4)
    safe = jnp.maximum(a, 1).astype(jnp.float32)
    t = jnp.log(safe / (NBUCKETS // 4))
    t = t / math.log(MAX_DISTANCE / (NBUCKETS // 4))
    t = t * (NBUCKETS // 2 - NBUCKETS // 4)
    large = (NBUCKETS // 4) + t.astype(jnp.int32)
    large = jnp.minimum(large, NBUCKETS // 2 - 1)
    bucket = neg + jnp.where(small, a, large)

    acc = jnp.zeros((H, LEXT), jnp.float32)
    for b in range(NBUCKETS):
        m = (bucket == b).astype(jnp.float32)
        acc = acc + table_t_ref[:, b:b + 1] * m
    for r in range(SH):
        out_ref[r] = acc[:, r:r + LW]


def _prep(table_t):
    return pl.pallas_call(
        _prep_body,
        out_shape=jax.ShapeDtypeStruct((SH, H, LW), jnp.float32),
    )(table_t)


_mesh = plsc.VectorSubcoreMesh(core_axis_name="c", subcore_axis_name="s")


@functools.partial(
    pl.kernel,
    out_type=jax.ShapeDtypeStruct((H, N, N), jnp.float32),
    mesh=_mesh,
    scratch_types=[
        pltpu.VMEM_SHARED((SH, H, LW), jnp.float32),
        pltpu.SemaphoreType.DMA,
        pltpu.SemaphoreType.DMA,
    ],
    compiler_params=pltpu.CompilerParams(use_tc_tiling_on_sc=False),
)
def _expand(line_hbm, out_hbm, shifts_sh, row_sem, load_sem):
    cid = lax.axis_index("c")
    sid = lax.axis_index("s")
    w = cid * 16 + sid

    @pl.when(sid == 0)
    def _load():
        pltpu.async_copy(line_hbm, shifts_sh, load_sem).wait()

    plsc.subcore_barrier()

    for g in range(_ROWS_PER_TILE // _FLIGHT):
        copies = []
        for k in range(_FLIGHT):
            i = w * _ROWS_PER_TILE + g * _FLIGHT + k
            start = (N - 1) - i
            r16 = jnp.bitwise_and(start, SH - 1)
            q = pl.multiple_of(start - r16, SH)
            copies.append(pltpu.async_copy(
                shifts_sh.at[r16, :, pl.ds(q, N)],
                out_hbm.at[:, i, :],
                row_sem,
            ))
        for cp in copies:
            cp.wait()


def kernel(n, relative_attention_bias):
    del n
    table_t = relative_attention_bias.T.astype(jnp.float32)
    line = _prep(table_t)
    return _expand(line)
